# baseline (speedup 1.0000x reference)
import functools

import jax
import jax.numpy as jnp
from jax import lax
from jax.experimental import pallas as pl
from jax.experimental.pallas import tpu as pltpu
from jax.experimental.pallas import tpu_sc as plsc

N = 10000
D = 128
N_PAD = 10240
E = 320000
ETOT = E + N
NTILES = 32
CHUNK = 64
NCHUNK = 162
EPT = NCHUNK * CHUNK
EP = EPT * NTILES
RPT = N_PAD // 16
BLK = 1280
NEG_SLOPE = 0.2


def _tc_feat_body(x_ref, w_ref, att_ref, h_ref, a_ref):
    h = jnp.dot(x_ref[...], w_ref[...], preferred_element_type=jnp.float32)
    h_ref[...] = h
    att = att_ref[...]
    a_s = jnp.sum(h * att[0:1, :], axis=1)
    a_d = jnp.sum(h * att[1:2, :], axis=1)
    a_ref[...] = jnp.concatenate(
        [a_s[None, :], a_d[None, :], jnp.zeros((6, a_s.shape[0]), jnp.float32)], 0
    )


def _tc_feat(x, W, att):
    return pl.pallas_call(
        _tc_feat_body,
        grid=(N_PAD // BLK,),
        in_specs=[
            pl.BlockSpec((BLK, D), lambda i: (i, 0)),
            pl.BlockSpec((D, D), lambda i: (0, 0)),
            pl.BlockSpec((8, D), lambda i: (0, 0)),
        ],
        out_specs=[
            pl.BlockSpec((BLK, D), lambda i: (i, 0)),
            pl.BlockSpec((8, BLK), lambda i: (0, i)),
        ],
        out_shape=[
            jax.ShapeDtypeStruct((N_PAD, D), jnp.float32),
            jax.ShapeDtypeStruct((8, N_PAD), jnp.float32),
        ],
    )(x, W, att)


def _tc_mid_body(accp_ref, denp_ref, b_ref, w_ref, att_ref, h_ref, a_ref):
    accp = accp_ref[...]
    acc = accp[0] + accp[1]
    denp = denp_ref[...]
    den = denp[0, 0] + denp[1, 0]
    x2 = acc / (den[:, None] + 1e-16) + b_ref[...][0:1, :]
    x2 = jnp.maximum(x2, 0.0)
    h = jnp.dot(x2, w_ref[...], preferred_element_type=jnp.float32)
    h_ref[...] = h
    att = att_ref[...]
    a_s = jnp.sum(h * att[0:1, :], axis=1)
    a_d = jnp.sum(h * att[1:2, :], axis=1)
    a_ref[...] = jnp.concatenate(
        [a_s[None, :], a_d[None, :], jnp.zeros((6, a_s.shape[0]), jnp.float32)], 0
    )


def _tc_mid(accp, denp, b, W, att):
    return pl.pallas_call(
        _tc_mid_body,
        grid=(N_PAD // BLK,),
        in_specs=[
            pl.BlockSpec((2, BLK, D), lambda i: (0, i, 0)),
            pl.BlockSpec((2, 8, BLK), lambda i: (0, 0, i)),
            pl.BlockSpec((8, D), lambda i: (0, 0)),
            pl.BlockSpec((D, D), lambda i: (0, 0)),
            pl.BlockSpec((8, D), lambda i: (0, 0)),
        ],
        out_specs=[
            pl.BlockSpec((BLK, D), lambda i: (i, 0)),
            pl.BlockSpec((8, BLK), lambda i: (0, i)),
        ],
        out_shape=[
            jax.ShapeDtypeStruct((N_PAD, D), jnp.float32),
            jax.ShapeDtypeStruct((8, N_PAD), jnp.float32),
        ],
    )(accp, denp, b, W, att)


def _tc_out_body(accp_ref, denp_ref, b_ref, o_ref):
    accp = accp_ref[...]
    acc = accp[0] + accp[1]
    denp = denp_ref[...]
    den = denp[0, 0] + denp[1, 0]
    o_ref[...] = acc / (den[:, None] + 1e-16) + b_ref[...][0:1, :]


def _tc_out(accp, denp, b):
    return pl.pallas_call(
        _tc_out_body,
        grid=(N_PAD // BLK,),
        in_specs=[
            pl.BlockSpec((2, BLK, D), lambda i: (0, i, 0)),
            pl.BlockSpec((2, 8, BLK), lambda i: (0, 0, i)),
            pl.BlockSpec((8, D), lambda i: (0, 0)),
        ],
        out_specs=pl.BlockSpec((BLK, D), lambda i: (i, 0)),
        out_shape=jax.ShapeDtypeStruct((N_PAD, D), jnp.float32),
    )(accp, denp, b)


@functools.cache
def _make_sc_edge():
    mesh = plsc.VectorSubcoreMesh(
        core_axis_name="c", subcore_axis_name="s", num_cores=2, num_subcores=16
    )
    GRP = CHUNK // 16

    def body(h_hbm, a_hbm, src_hbm, dst_hbm, acc_out, den_out,
             a_src_v, a_dst_v,
             src0, src1, src2, dst0, dst1, dst2,
             sd0, sd1, sd2, w0, w1, w2, rb0, rb1, rb2,
             acc_s, den_s,
             isem0, isem1, isem2, gsem0, gsem1, gsem2, ssem0, ssem1, ssem2):
        c = lax.axis_index("c")
        s = lax.axis_index("s")
        wid = c * 16 + s
        base = s * RPT
        src_r = (src0, src1, src2)
        dst_r = (dst0, dst1, dst2)
        sd_r = (sd0, sd1, sd2)
        w_r = (w0, w1, w2)
        rows_r = (rb0, rb1, rb2)
        isem = (isem0, isem1, isem2)
        gsem = (gsem0, gsem1, gsem2)
        ssem = (ssem0, ssem1, ssem2)

        def compute_w(b):
            for i in range(GRP):
                sidx = src_r[b][pl.ds(i * 16, 16)]
                didx = dst_r[b][pl.ds(i * 16, 16)]
                av = plsc.load_gather(a_src_v, [sidx]) + plsc.load_gather(a_dst_v, [didx])
                av = jnp.where(av >= 0.0, av, av * NEG_SLOPE)
                w_r[b][pl.ds(i * 16, 16)] = jnp.exp(av)
                sd_r[b][pl.ds(i * 16, 16)] = didx

        def scale_rows(b):
            rows_v = rows_r[b]
            for g in range(GRP):
                w16 = w_r[b][pl.ds(g * 16, 16)]
                splats = []
                for rr in range(16):
                    splats.append(lax.gather(
                        w16, jnp.full((16, 1), rr, jnp.int32),
                        lax.GatherDimensionNumbers(
                            offset_dims=(), collapsed_slice_dims=(0,),
                            start_index_map=(0,)),
                        slice_sizes=(1,),
                        mode=lax.GatherScatterMode.PROMISE_IN_BOUNDS))
                for rr in range(16):
                    r = g * 16 + rr
                    for i in range(8):
                        rows_v[r, pl.ds(i * 16, 16)] = rows_v[r, pl.ds(i * 16, 16)] * splats[rr]

        def issue_idx(j, b):
            pltpu.async_copy(src_hbm.at[wid, j], src_r[b], isem[b])
            pltpu.async_copy(dst_hbm.at[wid, j], dst_r[b], isem[b])

        def wait_idx(b):
            pltpu.make_async_copy(src_hbm.at[wid, 0], src_r[b], isem[b]).wait()
            pltpu.make_async_copy(src_hbm.at[wid, 0], dst_r[b], isem[b]).wait()

        def issue_gather(b):
            pltpu.async_copy(h_hbm.at[src_r[b]], rows_r[b], gsem[b])

        def wait_gather(b):
            pltpu.make_async_copy(h_hbm.at[src_r[b]], rows_r[b], gsem[b]).wait()

        def issue_scatter(b):
            pltpu.async_copy(rows_r[b], acc_s.at[sd_r[b]], ssem[b], add=True)
            pltpu.async_copy(w_r[b], den_s.at[sd_r[b]], ssem[b], add=True)

        def wait_scatter(b):
            pltpu.make_async_copy(rows_r[b], acc_s.at[sd_r[b]], ssem[b]).wait()
            pltpu.make_async_copy(w_r[b], den_s.at[sd_r[b]], ssem[b]).wait()

        pltpu.sync_copy(a_hbm.at[0], a_src_v)
        pltpu.sync_copy(a_hbm.at[1], a_dst_v)

        zero16 = jnp.zeros((16,), jnp.float32)
        for r in range(CHUNK):
            for i in range(8):
                rb0[r, pl.ds(i * 16, 16)] = zero16
        for q in range(RPT // CHUNK):
            pltpu.sync_copy(rb0, acc_s.at[pl.ds(base + q * CHUNK, CHUNK)])
        for q in range(RPT // 128):
            pltpu.sync_copy(rb0.at[0], den_s.at[pl.ds(base + q * 128, 128)])
        plsc.subcore_barrier()

        for k in range(3):
            issue_idx(k, k)
        wait_idx(0)
        issue_gather(0)
        compute_w(0)

        @pl.loop(0, NCHUNK, step=3)
        def outer(jj):
            for u in range(3):
                j = jj + u
                b = u
                b1 = (u + 1) % 3

                @pl.when(j + 1 < NCHUNK)
                def _():
                    wait_idx(b1)

                    @pl.when(j >= 2)
                    def _():
                        wait_scatter(b1)

                    issue_gather(b1)
                    compute_w(b1)

                wait_gather(b)

                @pl.when(j + 3 < NCHUNK)
                def _():
                    issue_idx(j + 3, b)

                scale_rows(b)
                issue_scatter(b)

        for b in range(3):
            wait_scatter(b)

        plsc.subcore_barrier()
        pltpu.sync_copy(acc_s.at[pl.ds(base, RPT)],
                        acc_out.at[c, pl.ds(base, RPT)])
        pltpu.sync_copy(den_s.at[pl.ds(base, RPT)],
                        den_out.at[c, 0, pl.ds(base, RPT)])

    return pl.kernel(
        body,
        out_type=[
            jax.ShapeDtypeStruct((2, N_PAD, D), jnp.float32),
            jax.ShapeDtypeStruct((2, 8, N_PAD), jnp.float32),
        ],
        mesh=mesh,
        compiler_params=pltpu.CompilerParams(needs_layout_passes=False),
        scratch_types=(
            [pltpu.VMEM((N_PAD,), jnp.float32)] * 2
            + [pltpu.VMEM((CHUNK,), jnp.int32)] * 9
            + [pltpu.VMEM((CHUNK,), jnp.float32)] * 3
            + [pltpu.VMEM((CHUNK, D), jnp.float32)] * 3
            + [pltpu.VMEM_SHARED((N_PAD, D), jnp.float32),
               pltpu.VMEM_SHARED((N_PAD,), jnp.float32)]
            + [pltpu.SemaphoreType.DMA] * 9
        ),
    )


def _att_rows(att_src, att_dst):
    return jnp.concatenate(
        [att_src.reshape(1, D), att_dst.reshape(1, D), jnp.zeros((6, D), jnp.float32)], 0
    )


def kernel(fea_mats, edge_index, W1, att_src1, att_dst1, b1,
           W2, att_src2, att_dst2, b2):
    x = jnp.pad(fea_mats[0], ((0, N_PAD - N), (0, 0)))
    loops = jnp.arange(N, dtype=jnp.int32)
    pad_idx = (jnp.arange(EP - ETOT, dtype=jnp.int32) % 64) + N
    src = jnp.concatenate([edge_index[0].astype(jnp.int32), loops, pad_idx])
    dst = jnp.concatenate([edge_index[1].astype(jnp.int32), loops, pad_idx])
    src = src.reshape(NTILES, NCHUNK, CHUNK)
    dst = dst.reshape(NTILES, NCHUNK, CHUNK)

    att1 = _att_rows(att_src1, att_dst1)
    att2 = _att_rows(att_src2, att_dst2)
    b1r = jnp.broadcast_to(b1.reshape(1, D), (8, D))
    b2r = jnp.broadcast_to(b2.reshape(1, D), (8, D))

    sc_edge = _make_sc_edge()
    h1, a1 = _tc_feat(x, W1, att1)
    accp1, denp1 = sc_edge(h1, a1, src, dst)
    h2, a2 = _tc_mid(accp1, denp1, b1r, W2, att2)
    accp2, denp2 = sc_edge(h2, a2, src, dst)
    out = _tc_out(accp2, denp2, b2r)
    return out[:N][None]


# speedup vs baseline: 1.0221x; 1.0221x over previous
import functools

import jax
import jax.numpy as jnp
from jax import lax
from jax.experimental import pallas as pl
from jax.experimental.pallas import tpu as pltpu
from jax.experimental.pallas import tpu_sc as plsc

N = 10000
D = 128
N_PAD = 10240
E = 320000
ETOT = E + N
NTILES = 32
CHUNK = 64
NCHUNK = 162
EPT = NCHUNK * CHUNK
EP = EPT * NTILES
RPT = N_PAD // 16
BLK = 1280
NEG_SLOPE = 0.2


def _tc_feat_body(x_ref, w_ref, att_ref, h_ref, a_ref):
    h = jnp.dot(x_ref[...], w_ref[...], preferred_element_type=jnp.float32)
    h_ref[...] = h
    att = att_ref[...]
    a_s = jnp.sum(h * att[0:1, :], axis=1)
    a_d = jnp.sum(h * att[1:2, :], axis=1)
    a_ref[...] = jnp.concatenate(
        [a_s[None, :], a_d[None, :], jnp.zeros((6, a_s.shape[0]), jnp.float32)], 0
    )


def _tc_feat(x, W, att):
    return pl.pallas_call(
        _tc_feat_body,
        grid=(N_PAD // BLK,),
        in_specs=[
            pl.BlockSpec((BLK, D), lambda i: (i, 0)),
            pl.BlockSpec((D, D), lambda i: (0, 0)),
            pl.BlockSpec((8, D), lambda i: (0, 0)),
        ],
        out_specs=[
            pl.BlockSpec((BLK, D), lambda i: (i, 0)),
            pl.BlockSpec((8, BLK), lambda i: (0, i)),
        ],
        out_shape=[
            jax.ShapeDtypeStruct((N_PAD, D), jnp.float32),
            jax.ShapeDtypeStruct((8, N_PAD), jnp.float32),
        ],
    )(x, W, att)


def _tc_mid_body(accp_ref, denp_ref, b_ref, w_ref, att_ref, h_ref, a_ref):
    accp = accp_ref[...]
    acc = accp[0] + accp[1]
    denp = denp_ref[...]
    den = denp[0, 0] + denp[1, 0]
    x2 = acc / (den[:, None] + 1e-16) + b_ref[...][0:1, :]
    x2 = jnp.maximum(x2, 0.0)
    h = jnp.dot(x2, w_ref[...], preferred_element_type=jnp.float32)
    h_ref[...] = h
    att = att_ref[...]
    a_s = jnp.sum(h * att[0:1, :], axis=1)
    a_d = jnp.sum(h * att[1:2, :], axis=1)
    a_ref[...] = jnp.concatenate(
        [a_s[None, :], a_d[None, :], jnp.zeros((6, a_s.shape[0]), jnp.float32)], 0
    )


def _tc_mid(accp, denp, b, W, att):
    return pl.pallas_call(
        _tc_mid_body,
        grid=(N_PAD // BLK,),
        in_specs=[
            pl.BlockSpec((2, BLK, D), lambda i: (0, i, 0)),
            pl.BlockSpec((2, 8, BLK), lambda i: (0, 0, i)),
            pl.BlockSpec((8, D), lambda i: (0, 0)),
            pl.BlockSpec((D, D), lambda i: (0, 0)),
            pl.BlockSpec((8, D), lambda i: (0, 0)),
        ],
        out_specs=[
            pl.BlockSpec((BLK, D), lambda i: (i, 0)),
            pl.BlockSpec((8, BLK), lambda i: (0, i)),
        ],
        out_shape=[
            jax.ShapeDtypeStruct((N_PAD, D), jnp.float32),
            jax.ShapeDtypeStruct((8, N_PAD), jnp.float32),
        ],
    )(accp, denp, b, W, att)


def _tc_out_body(accp_ref, denp_ref, b_ref, o_ref):
    accp = accp_ref[...]
    acc = accp[0] + accp[1]
    denp = denp_ref[...]
    den = denp[0, 0] + denp[1, 0]
    o_ref[...] = acc / (den[:, None] + 1e-16) + b_ref[...][0:1, :]


def _tc_out(accp, denp, b):
    return pl.pallas_call(
        _tc_out_body,
        grid=(N_PAD // BLK,),
        in_specs=[
            pl.BlockSpec((2, BLK, D), lambda i: (0, i, 0)),
            pl.BlockSpec((2, 8, BLK), lambda i: (0, 0, i)),
            pl.BlockSpec((8, D), lambda i: (0, 0)),
        ],
        out_specs=pl.BlockSpec((BLK, D), lambda i: (i, 0)),
        out_shape=jax.ShapeDtypeStruct((N_PAD, D), jnp.float32),
    )(accp, denp, b)


@functools.cache
def _make_sc_edge():
    mesh = plsc.VectorSubcoreMesh(
        core_axis_name="c", subcore_axis_name="s", num_cores=2, num_subcores=16
    )
    GRP = CHUNK // 16

    def body(h_hbm, a_hbm, src_hbm, dst_hbm, acc_out, den_out,
             a_src_v, a_dst_v,
             src0, src1, src2, dst0, dst1, dst2,
             sd0, sd1, sd2, w0, w1, w2, rb0, rb1, rb2,
             acc_s, den_s,
             isem0, isem1, isem2, gsem0, gsem1, gsem2, hsem0, hsem1, hsem2,
             ssem0, ssem1, ssem2):
        c = lax.axis_index("c")
        s = lax.axis_index("s")
        wid = c * 16 + s
        base = s * RPT
        src_r = (src0, src1, src2)
        dst_r = (dst0, dst1, dst2)
        sd_r = (sd0, sd1, sd2)
        w_r = (w0, w1, w2)
        rows_r = (rb0, rb1, rb2)
        isem = (isem0, isem1, isem2)
        gsem = (gsem0, gsem1, gsem2)
        hsem = (hsem0, hsem1, hsem2)
        ssem = (ssem0, ssem1, ssem2)

        def compute_w(b):
            for i in range(GRP):
                sidx = src_r[b][pl.ds(i * 16, 16)]
                didx = dst_r[b][pl.ds(i * 16, 16)]
                av = plsc.load_gather(a_src_v, [sidx]) + plsc.load_gather(a_dst_v, [didx])
                av = jnp.where(av >= 0.0, av, av * NEG_SLOPE)
                w_r[b][pl.ds(i * 16, 16)] = jnp.exp(av)
                sd_r[b][pl.ds(i * 16, 16)] = didx

        def scale_rows(b, h):
            rows_v = rows_r[b]
            for g in range(h * GRP // 2, (h + 1) * GRP // 2):
                w16 = w_r[b][pl.ds(g * 16, 16)]
                splats = []
                for rr in range(16):
                    splats.append(lax.gather(
                        w16, jnp.full((16, 1), rr, jnp.int32),
                        lax.GatherDimensionNumbers(
                            offset_dims=(), collapsed_slice_dims=(0,),
                            start_index_map=(0,)),
                        slice_sizes=(1,),
                        mode=lax.GatherScatterMode.PROMISE_IN_BOUNDS))
                for rr in range(16):
                    r = g * 16 + rr
                    for i in range(8):
                        rows_v[r, pl.ds(i * 16, 16)] = rows_v[r, pl.ds(i * 16, 16)] * splats[rr]

        def issue_idx(j, b):
            pltpu.async_copy(src_hbm.at[wid, j], src_r[b], isem[b])
            pltpu.async_copy(dst_hbm.at[wid, j], dst_r[b], isem[b])

        def wait_idx(b):
            pltpu.make_async_copy(src_hbm.at[wid, 0], src_r[b], isem[b]).wait()
            pltpu.make_async_copy(src_hbm.at[wid, 0], dst_r[b], isem[b]).wait()

        HALF = CHUNK // 2

        def issue_gather(b):
            pltpu.async_copy(h_hbm.at[src_r[b].at[pl.ds(0, HALF)]],
                             rows_r[b].at[pl.ds(0, HALF)], gsem[b])
            pltpu.async_copy(h_hbm.at[src_r[b].at[pl.ds(HALF, HALF)]],
                             rows_r[b].at[pl.ds(HALF, HALF)], hsem[b])

        def wait_gather_half(b, h):
            sem = gsem[b] if h == 0 else hsem[b]
            pltpu.make_async_copy(h_hbm.at[src_r[b].at[pl.ds(h * HALF, HALF)]],
                                  rows_r[b].at[pl.ds(h * HALF, HALF)], sem).wait()

        def issue_scatter(b):
            pltpu.async_copy(rows_r[b], acc_s.at[sd_r[b]], ssem[b], add=True)
            pltpu.async_copy(w_r[b], den_s.at[sd_r[b]], ssem[b], add=True)

        def wait_scatter(b):
            pltpu.make_async_copy(rows_r[b], acc_s.at[sd_r[b]], ssem[b]).wait()
            pltpu.make_async_copy(w_r[b], den_s.at[sd_r[b]], ssem[b]).wait()

        pltpu.sync_copy(a_hbm.at[0], a_src_v)
        pltpu.sync_copy(a_hbm.at[1], a_dst_v)

        zero16 = jnp.zeros((16,), jnp.float32)
        for r in range(CHUNK):
            for i in range(8):
                rb0[r, pl.ds(i * 16, 16)] = zero16
        for q in range(RPT // CHUNK):
            pltpu.sync_copy(rb0, acc_s.at[pl.ds(base + q * CHUNK, CHUNK)])
        for q in range(RPT // 128):
            pltpu.sync_copy(rb0.at[0], den_s.at[pl.ds(base + q * 128, 128)])
        plsc.subcore_barrier()

        for k in range(3):
            issue_idx(k, k)
        wait_idx(0)
        issue_gather(0)
        compute_w(0)

        @pl.loop(0, NCHUNK, step=3)
        def outer(jj):
            for u in range(3):
                j = jj + u
                b = u
                b1 = (u + 1) % 3

                @pl.when(j + 1 < NCHUNK)
                def _():
                    wait_idx(b1)

                    @pl.when(j >= 2)
                    def _():
                        wait_scatter(b1)

                    issue_gather(b1)
                    compute_w(b1)

                wait_gather_half(b, 0)

                @pl.when(j + 3 < NCHUNK)
                def _():
                    issue_idx(j + 3, b)

                scale_rows(b, 0)
                wait_gather_half(b, 1)
                scale_rows(b, 1)
                issue_scatter(b)

        for b in range(3):
            wait_scatter(b)

        plsc.subcore_barrier()
        pltpu.sync_copy(acc_s.at[pl.ds(base, RPT)],
                        acc_out.at[c, pl.ds(base, RPT)])
        pltpu.sync_copy(den_s.at[pl.ds(base, RPT)],
                        den_out.at[c, 0, pl.ds(base, RPT)])

    return pl.kernel(
        body,
        out_type=[
            jax.ShapeDtypeStruct((2, N_PAD, D), jnp.float32),
            jax.ShapeDtypeStruct((2, 8, N_PAD), jnp.float32),
        ],
        mesh=mesh,
        compiler_params=pltpu.CompilerParams(needs_layout_passes=False),
        scratch_types=(
            [pltpu.VMEM((N_PAD,), jnp.float32)] * 2
            + [pltpu.VMEM((CHUNK,), jnp.int32)] * 9
            + [pltpu.VMEM((CHUNK,), jnp.float32)] * 3
            + [pltpu.VMEM((CHUNK, D), jnp.float32)] * 3
            + [pltpu.VMEM_SHARED((N_PAD, D), jnp.float32),
               pltpu.VMEM_SHARED((N_PAD,), jnp.float32)]
            + [pltpu.SemaphoreType.DMA] * 12
        ),
    )


def _att_rows(att_src, att_dst):
    return jnp.concatenate(
        [att_src.reshape(1, D), att_dst.reshape(1, D), jnp.zeros((6, D), jnp.float32)], 0
    )


def kernel(fea_mats, edge_index, W1, att_src1, att_dst1, b1,
           W2, att_src2, att_dst2, b2):
    x = jnp.pad(fea_mats[0], ((0, N_PAD - N), (0, 0)))
    loops = jnp.arange(N, dtype=jnp.int32)
    pad_idx = (jnp.arange(EP - ETOT, dtype=jnp.int32) % 64) + N
    src = jnp.concatenate([edge_index[0].astype(jnp.int32), loops, pad_idx])
    dst = jnp.concatenate([edge_index[1].astype(jnp.int32), loops, pad_idx])
    src = src.reshape(NTILES, NCHUNK, CHUNK)
    dst = dst.reshape(NTILES, NCHUNK, CHUNK)

    att1 = _att_rows(att_src1, att_dst1)
    att2 = _att_rows(att_src2, att_dst2)
    b1r = jnp.broadcast_to(b1.reshape(1, D), (8, D))
    b2r = jnp.broadcast_to(b2.reshape(1, D), (8, D))

    sc_edge = _make_sc_edge()
    h1, a1 = _tc_feat(x, W1, att1)
    accp1, denp1 = sc_edge(h1, a1, src, dst)
    h2, a2 = _tc_mid(accp1, denp1, b1r, W2, att2)
    accp2, denp2 = sc_edge(h2, a2, src, dst)
    out = _tc_out(accp2, denp2, b2r)
    return out[:N][None]
